# Initial kernel scaffold; baseline (speedup 1.0000x reference)
import jax
import jax.numpy as jnp
from jax.experimental import pallas as pl


def kernel(x, edge_index, batch, W1, b1, W2, b2, Wout, bout):
    raise NotImplementedError("write your pallas kernel here")


# speedup vs baseline: 17.0450x; 17.0450x over previous
import functools

import jax
import jax.numpy as jnp
from jax import lax
from jax.experimental import pallas as pl
from jax.experimental.pallas import tpu as pltpu
from jax.experimental.pallas import tpu_sc as plsc

N_NODES = 10000
N_EDGES = 320000
D_FEAT = 128
HIDDEN = 64
N_GRAPHS = 64

NC = 2
NS = 16
NW = NC * NS
CHUNK = 128
EPT = N_EDGES // NW
NCH = (EPT + CHUNK - 1) // CHUNK
E_PAD = NW * NCH * CHUNK
ROWS_PER_TILE = 632
N_PAD = ROWS_PER_TILE * NS

_MESH = plsc.VectorSubcoreMesh(
    core_axis_name="c", subcore_axis_name="s", num_cores=NC, num_subcores=NS)


def _deg_body(dst3, ones_v_hbm, zeros16, out, ones_v, dst_v, deg_sh):
  cid = lax.axis_index("c")
  sid = lax.axis_index("s")
  wid = cid * NS + sid
  r0 = sid * ROWS_PER_TILE
  pltpu.sync_copy(zeros16.at[pl.ds(r0, ROWS_PER_TILE)],
                  deg_sh.at[pl.ds(r0, ROWS_PER_TILE)])
  pltpu.sync_copy(ones_v_hbm, ones_v)
  pltpu.sync_copy(dst3.at[wid], dst_v)
  plsc.subcore_barrier()

  @pl.loop(0, NCH)
  def _(j):
    pltpu.sync_copy(ones_v, deg_sh.at[dst_v.at[j]], add=True)

  plsc.subcore_barrier()
  pltpu.sync_copy(deg_sh.at[pl.ds(r0, ROWS_PER_TILE)],
                  out.at[cid, pl.ds(r0, ROWS_PER_TILE)])


_SC_PARAMS = pltpu.CompilerParams(use_tc_tiling_on_sc=False)

_deg_call = pl.kernel(
    _deg_body,
    out_type=jax.ShapeDtypeStruct((NC, N_PAD, 16), jnp.float32),
    mesh=_MESH,
    compiler_params=_SC_PARAMS,
    scratch_types=[
        pltpu.VMEM((CHUNK, 16), jnp.float32),
        pltpu.VMEM((NCH, CHUNK), jnp.int32),
        pltpu.VMEM_SHARED((N_PAD, 16), jnp.float32),
    ],
)


def _agg_body(h_hbm, src3, dst3, zeros64, out, idx_v, dst_v, rows_v, acc_sh,
              sem):
  cid = lax.axis_index("c")
  sid = lax.axis_index("s")
  wid = cid * NS + sid
  r0 = sid * ROWS_PER_TILE
  pltpu.sync_copy(zeros64.at[pl.ds(r0, ROWS_PER_TILE)],
                  acc_sh.at[pl.ds(r0, ROWS_PER_TILE)])
  pltpu.sync_copy(dst3.at[wid], dst_v)
  plsc.subcore_barrier()

  @pl.loop(0, NCH)
  def _(j):
    pltpu.sync_copy(src3.at[wid, j], idx_v)
    pltpu.async_copy(h_hbm.at[idx_v], rows_v, sem).wait()
    pltpu.sync_copy(rows_v, acc_sh.at[dst_v.at[j]], add=True)

  plsc.subcore_barrier()
  pltpu.sync_copy(acc_sh.at[pl.ds(r0, ROWS_PER_TILE)],
                  out.at[cid, pl.ds(r0, ROWS_PER_TILE)])


_agg_call = pl.kernel(
    _agg_body,
    out_type=jax.ShapeDtypeStruct((NC, N_PAD, HIDDEN), jnp.float32),
    mesh=_MESH,
    compiler_params=_SC_PARAMS,
    scratch_types=[
        pltpu.VMEM((CHUNK,), jnp.int32),
        pltpu.VMEM((NCH, CHUNK), jnp.int32),
        pltpu.VMEM((CHUNK, HIDDEN), jnp.float32),
        pltpu.VMEM_SHARED((N_PAD, HIDDEN), jnp.float32),
        pltpu.SemaphoreType.DMA,
    ],
)


def _tc1_body(degp, x_ref, w_ref, h_out, dis_out):
  deg = degp[0, :, 0:1] + degp[1, :, 0:1] + 1.0
  dis = lax.rsqrt(deg)
  h = jnp.dot(x_ref[...], w_ref[...], preferred_element_type=jnp.float32)
  h_out[...] = h * dis
  dis_out[...] = dis


def _tc2_body(part, hs_ref, dis_ref, b_ref, w_ref, out_ref):
  dis = dis_ref[...]
  s = part[0] + part[1] + hs_ref[...]
  h = jnp.maximum(dis * s + b_ref[...], 0.0)
  out_ref[...] = jnp.dot(
      h, w_ref[...], preferred_element_type=jnp.float32) * dis


def _tc3_body(part, hs_ref, dis_ref, b_ref, batch_ref, wout_ref, bout_ref,
              out_ref):
  dis = dis_ref[...]
  s = part[0] + part[1] + hs_ref[...]
  h2 = jnp.maximum(dis * s + b_ref[...], 0.0)
  gids = lax.broadcasted_iota(jnp.int32, (N_GRAPHS, N_PAD), 0)
  mask = (gids == batch_ref[...]).astype(jnp.float32)
  sums = jnp.dot(mask, h2, preferred_element_type=jnp.float32)
  cnt = jnp.sum(mask, axis=1, keepdims=True)
  g = sums / jnp.maximum(cnt, 1.0)
  out_ref[...] = jnp.dot(
      g, wout_ref[...], preferred_element_type=jnp.float32) + bout_ref[...]


def kernel(x, edge_index, batch, W1, b1, W2, b2, Wout, bout):
  src = edge_index[0].astype(jnp.int32)
  dst = edge_index[1].astype(jnp.int32)
  pad = E_PAD - N_EDGES
  src3 = jnp.concatenate(
      [src, jnp.full((pad,), N_NODES, jnp.int32)]).reshape(NW, NCH, CHUNK)
  dst3 = jnp.concatenate(
      [dst, jnp.full((pad,), N_NODES, jnp.int32)]).reshape(NW, NCH, CHUNK)

  xp = jnp.zeros((N_PAD, D_FEAT), x.dtype).at[:N_NODES].set(x)
  zeros16 = jnp.zeros((N_PAD, 16), jnp.float32)
  zeros64 = jnp.zeros((N_PAD, HIDDEN), jnp.float32)
  ones_v = jnp.ones((CHUNK, 16), jnp.float32)
  batch_p = jnp.full((1, N_PAD), N_GRAPHS, jnp.int32).at[0, :N_NODES].set(
      batch.astype(jnp.int32))
  wout_p = jnp.zeros((HIDDEN, 128), jnp.float32).at[:, :2].set(Wout)
  bout_p = jnp.zeros((1, 128), jnp.float32).at[0, :2].set(bout)

  degp = _deg_call(dst3, ones_v, zeros16)

  h1s, dis = pl.pallas_call(
      _tc1_body,
      out_shape=(
          jax.ShapeDtypeStruct((N_PAD, HIDDEN), jnp.float32),
          jax.ShapeDtypeStruct((N_PAD, 1), jnp.float32),
      ),
  )(degp, xp, W1)

  part1 = _agg_call(h1s, src3, dst3, zeros64)

  h2s = pl.pallas_call(
      _tc2_body,
      out_shape=jax.ShapeDtypeStruct((N_PAD, HIDDEN), jnp.float32),
  )(part1, h1s, dis, b1.reshape(1, HIDDEN), W2)

  part2 = _agg_call(h2s, src3, dst3, zeros64)

  outp = pl.pallas_call(
      _tc3_body,
      out_shape=jax.ShapeDtypeStruct((N_GRAPHS, 128), jnp.float32),
  )(part2, h2s, dis, b2.reshape(1, HIDDEN), batch_p, wout_p, bout_p)

  return outp[:, :2]


# speedup vs baseline: 39.1517x; 2.2970x over previous
import functools

import jax
import jax.numpy as jnp
from jax import lax
from jax.experimental import pallas as pl
from jax.experimental.pallas import tpu as pltpu
from jax.experimental.pallas import tpu_sc as plsc

N_NODES = 10000
N_EDGES = 320000
D_FEAT = 128
HIDDEN = 64
N_GRAPHS = 64

NC = 2
NS = 16
NW = NC * NS
CHUNK = 128
EPT = N_EDGES // NW
NCH = 80
E_PAD = NW * NCH * CHUNK
ROWS_PER_TILE = 632
N_PAD = ROWS_PER_TILE * NS

_MESH = plsc.VectorSubcoreMesh(
    core_axis_name="c", subcore_axis_name="s", num_cores=NC, num_subcores=NS)


def _deg_body(dst3, ones_v_hbm, zeros16, out, ones_v, dst_v, deg_sh):
  cid = lax.axis_index("c")
  sid = lax.axis_index("s")
  wid = cid * NS + sid
  r0 = sid * ROWS_PER_TILE
  pltpu.sync_copy(zeros16.at[pl.ds(r0, ROWS_PER_TILE)],
                  deg_sh.at[pl.ds(r0, ROWS_PER_TILE)])
  pltpu.sync_copy(ones_v_hbm, ones_v)
  pltpu.sync_copy(dst3.at[wid], dst_v)
  plsc.subcore_barrier()

  @pl.loop(0, NCH)
  def _(j):
    pltpu.sync_copy(ones_v, deg_sh.at[dst_v.at[j]], add=True)

  plsc.subcore_barrier()
  pltpu.sync_copy(deg_sh.at[pl.ds(r0, ROWS_PER_TILE)],
                  out.at[cid, pl.ds(r0, ROWS_PER_TILE)])


_SC_PARAMS = pltpu.CompilerParams(use_tc_tiling_on_sc=False)

_deg_call = pl.kernel(
    _deg_body,
    out_type=jax.ShapeDtypeStruct((NC, N_PAD, 16), jnp.float32),
    mesh=_MESH,
    compiler_params=_SC_PARAMS,
    scratch_types=[
        pltpu.VMEM((CHUNK, 16), jnp.float32),
        pltpu.VMEM((NCH, CHUNK), jnp.int32),
        pltpu.VMEM_SHARED((N_PAD, 16), jnp.float32),
    ],
)


def _agg_body(h_hbm, src3, dst3, zeros64, out, src_v, dst_v, rows0, rows1,
              acc_sh, sem0, sem1):
  cid = lax.axis_index("c")
  sid = lax.axis_index("s")
  wid = cid * NS + sid
  r0 = sid * ROWS_PER_TILE
  pltpu.sync_copy(zeros64.at[pl.ds(r0, ROWS_PER_TILE)],
                  acc_sh.at[pl.ds(r0, ROWS_PER_TILE)])
  pltpu.sync_copy(src3.at[wid], src_v)
  pltpu.sync_copy(dst3.at[wid], dst_v)
  plsc.subcore_barrier()

  pltpu.async_copy(h_hbm.at[src_v.at[0]], rows0, sem0)

  @pl.loop(0, NCH, step=2)
  def _(j):
    pltpu.async_copy(h_hbm.at[src_v.at[j + 1]], rows1, sem1)
    pltpu.make_async_copy(h_hbm.at[src_v.at[j]], rows0, sem0).wait()
    pltpu.sync_copy(rows0, acc_sh.at[dst_v.at[j]], add=True)

    @pl.when(j + 2 < NCH)
    def _():
      pltpu.async_copy(h_hbm.at[src_v.at[j + 2]], rows0, sem0)

    pltpu.make_async_copy(h_hbm.at[src_v.at[j + 1]], rows1, sem1).wait()
    pltpu.sync_copy(rows1, acc_sh.at[dst_v.at[j + 1]], add=True)

  plsc.subcore_barrier()
  pltpu.sync_copy(acc_sh.at[pl.ds(r0, ROWS_PER_TILE)],
                  out.at[cid, pl.ds(r0, ROWS_PER_TILE)])


_agg_call = pl.kernel(
    _agg_body,
    out_type=jax.ShapeDtypeStruct((NC, N_PAD, HIDDEN), jnp.float32),
    mesh=_MESH,
    compiler_params=_SC_PARAMS,
    scratch_types=[
        pltpu.VMEM((NCH, CHUNK), jnp.int32),
        pltpu.VMEM((NCH, CHUNK), jnp.int32),
        pltpu.VMEM((CHUNK, HIDDEN), jnp.float32),
        pltpu.VMEM((CHUNK, HIDDEN), jnp.float32),
        pltpu.VMEM_SHARED((N_PAD, HIDDEN), jnp.float32),
        pltpu.SemaphoreType.DMA,
        pltpu.SemaphoreType.DMA,
    ],
)


def _tc1_body(degp, x_ref, w_ref, h_out, dis_out):
  deg = degp[0, :, 0:1] + degp[1, :, 0:1] + 1.0
  dis = lax.rsqrt(deg)
  h = jnp.dot(x_ref[...], w_ref[...], preferred_element_type=jnp.float32)
  h_out[...] = h * dis
  dis_out[...] = dis


def _tc2_body(part, hs_ref, dis_ref, b_ref, w_ref, out_ref):
  dis = dis_ref[...]
  s = part[0] + part[1] + hs_ref[...]
  h = jnp.maximum(dis * s + b_ref[...], 0.0)
  out_ref[...] = jnp.dot(
      h, w_ref[...], preferred_element_type=jnp.float32) * dis


def _tc3_body(part, hs_ref, dis_ref, b_ref, batch_ref, wout_ref, bout_ref,
              out_ref):
  dis = dis_ref[...]
  s = part[0] + part[1] + hs_ref[...]
  h2 = jnp.maximum(dis * s + b_ref[...], 0.0)
  gids = lax.broadcasted_iota(jnp.int32, (N_GRAPHS, N_PAD), 0)
  mask = (gids == batch_ref[...]).astype(jnp.float32)
  sums = jnp.dot(mask, h2, preferred_element_type=jnp.float32)
  cnt = jnp.sum(mask, axis=1, keepdims=True)
  g = sums / jnp.maximum(cnt, 1.0)
  out_ref[...] = jnp.dot(
      g, wout_ref[...], preferred_element_type=jnp.float32) + bout_ref[...]


def kernel(x, edge_index, batch, W1, b1, W2, b2, Wout, bout):
  src = edge_index[0].astype(jnp.int32)
  dst = edge_index[1].astype(jnp.int32)
  pad = E_PAD - N_EDGES
  pad_idx = N_NODES + (jnp.arange(pad, dtype=jnp.int32) % (N_PAD - N_NODES))
  src3 = jnp.concatenate([src, pad_idx]).reshape(NW, NCH, CHUNK)
  dst3 = jnp.concatenate([dst, pad_idx]).reshape(NW, NCH, CHUNK)

  xp = jnp.zeros((N_PAD, D_FEAT), x.dtype).at[:N_NODES].set(x)
  zeros16 = jnp.zeros((N_PAD, 16), jnp.float32)
  zeros64 = jnp.zeros((N_PAD, HIDDEN), jnp.float32)
  ones_v = jnp.ones((CHUNK, 16), jnp.float32)
  batch_p = jnp.full((1, N_PAD), N_GRAPHS, jnp.int32).at[0, :N_NODES].set(
      batch.astype(jnp.int32))
  wout_p = jnp.zeros((HIDDEN, 128), jnp.float32).at[:, :2].set(Wout)
  bout_p = jnp.zeros((1, 128), jnp.float32).at[0, :2].set(bout)

  degp = _deg_call(dst3, ones_v, zeros16)

  h1s, dis = pl.pallas_call(
      _tc1_body,
      out_shape=(
          jax.ShapeDtypeStruct((N_PAD, HIDDEN), jnp.float32),
          jax.ShapeDtypeStruct((N_PAD, 1), jnp.float32),
      ),
  )(degp, xp, W1)

  part1 = _agg_call(h1s, src3, dst3, zeros64)

  h2s = pl.pallas_call(
      _tc2_body,
      out_shape=jax.ShapeDtypeStruct((N_PAD, HIDDEN), jnp.float32),
  )(part1, h1s, dis, b1.reshape(1, HIDDEN), W2)

  part2 = _agg_call(h2s, src3, dst3, zeros64)

  outp = pl.pallas_call(
      _tc3_body,
      out_shape=jax.ShapeDtypeStruct((N_GRAPHS, 128), jnp.float32),
  )(part2, h2s, dis, b2.reshape(1, HIDDEN), batch_p, wout_p, bout_p)

  return outp[:, :2]


# speedup vs baseline: 43.2779x; 1.1054x over previous
import functools

import jax
import jax.numpy as jnp
from jax import lax
from jax.experimental import pallas as pl
from jax.experimental.pallas import tpu as pltpu
from jax.experimental.pallas import tpu_sc as plsc

N_NODES = 10000
N_EDGES = 320000
D_FEAT = 128
HIDDEN = 64
N_GRAPHS = 64

NC = 2
NS = 16
NW = NC * NS
CHUNK = 128
EPT = N_EDGES // NW
NCH = 80
E_PAD = NW * NCH * CHUNK
ROWS_PER_TILE = 632
N_PAD = ROWS_PER_TILE * NS

_MESH = plsc.VectorSubcoreMesh(
    core_axis_name="c", subcore_axis_name="s", num_cores=NC, num_subcores=NS)


def _deg_body(dst3, ones_v_hbm, zeros16, out, ones_v, dst_v, deg_sh, sem):
  cid = lax.axis_index("c")
  sid = lax.axis_index("s")
  wid = cid * NS + sid
  r0 = sid * ROWS_PER_TILE
  pltpu.sync_copy(zeros16.at[pl.ds(r0, ROWS_PER_TILE)],
                  deg_sh.at[pl.ds(r0, ROWS_PER_TILE)])
  pltpu.sync_copy(ones_v_hbm, ones_v)
  pltpu.sync_copy(dst3.at[wid], dst_v)
  plsc.subcore_barrier()

  def _deg_desc(j):
    return pltpu.make_async_copy(ones_v, deg_sh.at[dst_v.at[j]], sem)

  @pl.loop(0, NCH)
  def _(j):
    pltpu.async_copy(ones_v, deg_sh.at[dst_v.at[j]], sem, add=True)

  @pl.loop(0, NCH)
  def _(j):
    _deg_desc(j).wait()

  plsc.subcore_barrier()
  pltpu.sync_copy(deg_sh.at[pl.ds(r0, ROWS_PER_TILE)],
                  out.at[cid, pl.ds(r0, ROWS_PER_TILE)])


_SC_PARAMS = pltpu.CompilerParams(use_tc_tiling_on_sc=False)

_deg_call = pl.kernel(
    _deg_body,
    out_type=jax.ShapeDtypeStruct((NC, N_PAD, 16), jnp.float32),
    mesh=_MESH,
    compiler_params=_SC_PARAMS,
    scratch_types=[
        pltpu.VMEM((CHUNK, 16), jnp.float32),
        pltpu.VMEM((NCH, CHUNK), jnp.int32),
        pltpu.VMEM_SHARED((N_PAD, 16), jnp.float32),
        pltpu.SemaphoreType.DMA,
    ],
)


def _agg_body(h_hbm, src3, dst3, zeros64, out, src_v, dst_v, rows0, rows1,
              rows2, rows3, acc_sh, gsem0, gsem1, gsem2, gsem3, ssem0, ssem1,
              ssem2, ssem3):
  cid = lax.axis_index("c")
  sid = lax.axis_index("s")
  wid = cid * NS + sid
  r0 = sid * ROWS_PER_TILE
  pltpu.sync_copy(zeros64.at[pl.ds(r0, ROWS_PER_TILE)],
                  acc_sh.at[pl.ds(r0, ROWS_PER_TILE)])
  pltpu.sync_copy(src3.at[wid], src_v)
  pltpu.sync_copy(dst3.at[wid], dst_v)
  plsc.subcore_barrier()

  rows = (rows0, rows1, rows2, rows3)
  gsem = (gsem0, gsem1, gsem2, gsem3)
  ssem = (ssem0, ssem1, ssem2, ssem3)

  def _gather(k, b):
    pltpu.async_copy(h_hbm.at[src_v.at[k]], rows[b], gsem[b])

  def _scat_desc(k, b):
    return pltpu.make_async_copy(rows[b], acc_sh.at[dst_v.at[k]], ssem[b])

  _gather(0, 0)
  _gather(1, 1)

  @pl.loop(0, NCH, step=4)
  def _(j):
    for b in range(4):
      k = j + b

      @pl.when(k >= 2)
      def _():
        _scat_desc(k - 2, (b + 2) % 4).wait()

      @pl.when(k + 2 < NCH)
      def _():
        _gather(k + 2, (b + 2) % 4)

      pltpu.make_async_copy(h_hbm.at[src_v.at[k]], rows[b], gsem[b]).wait()
      pltpu.async_copy(rows[b], acc_sh.at[dst_v.at[k]], ssem[b], add=True)

  _scat_desc(NCH - 2, (NCH - 2) % 4).wait()
  _scat_desc(NCH - 1, (NCH - 1) % 4).wait()
  plsc.subcore_barrier()
  pltpu.sync_copy(acc_sh.at[pl.ds(r0, ROWS_PER_TILE)],
                  out.at[cid, pl.ds(r0, ROWS_PER_TILE)])


_agg_call = pl.kernel(
    _agg_body,
    out_type=jax.ShapeDtypeStruct((NC, N_PAD, HIDDEN), jnp.float32),
    mesh=_MESH,
    compiler_params=_SC_PARAMS,
    scratch_types=[
        pltpu.VMEM((NCH, CHUNK), jnp.int32),
        pltpu.VMEM((NCH, CHUNK), jnp.int32),
        pltpu.VMEM((CHUNK, HIDDEN), jnp.float32),
        pltpu.VMEM((CHUNK, HIDDEN), jnp.float32),
        pltpu.VMEM((CHUNK, HIDDEN), jnp.float32),
        pltpu.VMEM((CHUNK, HIDDEN), jnp.float32),
        pltpu.VMEM_SHARED((N_PAD, HIDDEN), jnp.float32),
    ] + [pltpu.SemaphoreType.DMA] * 8,
)


def _tc1_body(degp, x_ref, w_ref, h_out, dis_out):
  deg = degp[0, :, 0:1] + degp[1, :, 0:1] + 1.0
  dis = lax.rsqrt(deg)
  h = jnp.dot(x_ref[...], w_ref[...], preferred_element_type=jnp.float32)
  h_out[...] = h * dis
  dis_out[...] = dis


def _tc2_body(part, hs_ref, dis_ref, b_ref, w_ref, out_ref):
  dis = dis_ref[...]
  s = part[0] + part[1] + hs_ref[...]
  h = jnp.maximum(dis * s + b_ref[...], 0.0)
  out_ref[...] = jnp.dot(
      h, w_ref[...], preferred_element_type=jnp.float32) * dis


def _tc3_body(part, hs_ref, dis_ref, b_ref, batch_ref, wout_ref, bout_ref,
              out_ref):
  dis = dis_ref[...]
  s = part[0] + part[1] + hs_ref[...]
  h2 = jnp.maximum(dis * s + b_ref[...], 0.0)
  gids = lax.broadcasted_iota(jnp.int32, (N_GRAPHS, N_PAD), 0)
  mask = (gids == batch_ref[...]).astype(jnp.float32)
  sums = jnp.dot(mask, h2, preferred_element_type=jnp.float32)
  cnt = jnp.sum(mask, axis=1, keepdims=True)
  g = sums / jnp.maximum(cnt, 1.0)
  out_ref[...] = jnp.dot(
      g, wout_ref[...], preferred_element_type=jnp.float32) + bout_ref[...]


def kernel(x, edge_index, batch, W1, b1, W2, b2, Wout, bout):
  src = edge_index[0].astype(jnp.int32)
  dst = edge_index[1].astype(jnp.int32)
  pad = E_PAD - N_EDGES
  pad_idx = N_NODES + (jnp.arange(pad, dtype=jnp.int32) % (N_PAD - N_NODES))
  src3 = jnp.concatenate([src, pad_idx]).reshape(NW, NCH, CHUNK)
  dst3 = jnp.concatenate([dst, pad_idx]).reshape(NW, NCH, CHUNK)

  xp = jnp.zeros((N_PAD, D_FEAT), x.dtype).at[:N_NODES].set(x)
  zeros16 = jnp.zeros((N_PAD, 16), jnp.float32)
  zeros64 = jnp.zeros((N_PAD, HIDDEN), jnp.float32)
  ones_v = jnp.ones((CHUNK, 16), jnp.float32)
  batch_p = jnp.full((1, N_PAD), N_GRAPHS, jnp.int32).at[0, :N_NODES].set(
      batch.astype(jnp.int32))
  wout_p = jnp.zeros((HIDDEN, 128), jnp.float32).at[:, :2].set(Wout)
  bout_p = jnp.zeros((1, 128), jnp.float32).at[0, :2].set(bout)

  degp = _deg_call(dst3, ones_v, zeros16)

  h1s, dis = pl.pallas_call(
      _tc1_body,
      out_shape=(
          jax.ShapeDtypeStruct((N_PAD, HIDDEN), jnp.float32),
          jax.ShapeDtypeStruct((N_PAD, 1), jnp.float32),
      ),
  )(degp, xp, W1)

  part1 = _agg_call(h1s, src3, dst3, zeros64)

  h2s = pl.pallas_call(
      _tc2_body,
      out_shape=jax.ShapeDtypeStruct((N_PAD, HIDDEN), jnp.float32),
  )(part1, h1s, dis, b1.reshape(1, HIDDEN), W2)

  part2 = _agg_call(h2s, src3, dst3, zeros64)

  outp = pl.pallas_call(
      _tc3_body,
      out_shape=jax.ShapeDtypeStruct((N_GRAPHS, 128), jnp.float32),
  )(part2, h2s, dis, b2.reshape(1, HIDDEN), batch_p, wout_p, bout_p)

  return outp[:, :2]


# speedup vs baseline: 44.5375x; 1.0291x over previous
import functools

import jax
import jax.numpy as jnp
from jax import lax
from jax.experimental import pallas as pl
from jax.experimental.pallas import tpu as pltpu
from jax.experimental.pallas import tpu_sc as plsc

N_NODES = 10000
N_EDGES = 320000
D_FEAT = 128
HIDDEN = 64
N_GRAPHS = 64

NC = 2
NS = 16
NW = NC * NS
CHUNK = 128
EPT = N_EDGES // NW
NCH = 80
E_PAD = NW * NCH * CHUNK
RING = 8
ROWS_PER_TILE = 632
N_PAD = ROWS_PER_TILE * NS

_MESH = plsc.VectorSubcoreMesh(
    core_axis_name="c", subcore_axis_name="s", num_cores=NC, num_subcores=NS)


def _deg_body(dst3, ones_v_hbm, zeros16, out, ones_v, dst_v, deg_sh, sem):
  cid = lax.axis_index("c")
  sid = lax.axis_index("s")
  wid = cid * NS + sid
  r0 = sid * ROWS_PER_TILE
  pltpu.sync_copy(zeros16.at[pl.ds(r0, ROWS_PER_TILE)],
                  deg_sh.at[pl.ds(r0, ROWS_PER_TILE)])
  pltpu.sync_copy(ones_v_hbm, ones_v)
  pltpu.sync_copy(dst3.at[wid], dst_v)
  plsc.subcore_barrier()

  def _deg_desc(j):
    return pltpu.make_async_copy(ones_v, deg_sh.at[dst_v.at[j]], sem)

  @pl.loop(0, NCH)
  def _(j):
    pltpu.async_copy(ones_v, deg_sh.at[dst_v.at[j]], sem, add=True)

  @pl.loop(0, NCH)
  def _(j):
    _deg_desc(j).wait()

  plsc.subcore_barrier()
  pltpu.sync_copy(deg_sh.at[pl.ds(r0, ROWS_PER_TILE)],
                  out.at[cid, pl.ds(r0, ROWS_PER_TILE)])


_SC_PARAMS = pltpu.CompilerParams(use_tc_tiling_on_sc=False)

_deg_call = pl.kernel(
    _deg_body,
    out_type=jax.ShapeDtypeStruct((NC, N_PAD, 16), jnp.float32),
    mesh=_MESH,
    compiler_params=_SC_PARAMS,
    scratch_types=[
        pltpu.VMEM((CHUNK, 16), jnp.float32),
        pltpu.VMEM((NCH, CHUNK), jnp.int32),
        pltpu.VMEM_SHARED((N_PAD, 16), jnp.float32),
        pltpu.SemaphoreType.DMA,
    ],
)


def _agg_body(h_hbm, src3, dst3, zeros64, out, src_v, dst_v, *bufs):
  rows = bufs[:RING]
  acc_sh = bufs[RING]
  (gsem0, gsem1, gsem2, gsem3, gsem4, gsem5, gsem6, gsem7,
   ssem0, ssem1, ssem2, ssem3, ssem4, ssem5, ssem6, ssem7) = bufs[RING + 1:]
  cid = lax.axis_index("c")
  sid = lax.axis_index("s")
  wid = cid * NS + sid
  r0 = sid * ROWS_PER_TILE
  pltpu.sync_copy(zeros64.at[pl.ds(r0, ROWS_PER_TILE)],
                  acc_sh.at[pl.ds(r0, ROWS_PER_TILE)])
  pltpu.sync_copy(src3.at[wid], src_v)
  pltpu.sync_copy(dst3.at[wid], dst_v)
  plsc.subcore_barrier()

  gsem = (gsem0, gsem1, gsem2, gsem3, gsem4, gsem5, gsem6, gsem7)
  ssem = (ssem0, ssem1, ssem2, ssem3, ssem4, ssem5, ssem6, ssem7)
  H = RING // 2

  def _gather(k, b):
    pltpu.async_copy(h_hbm.at[src_v.at[k]], rows[b], gsem[b])

  def _scat_desc(k, b):
    return pltpu.make_async_copy(rows[b], acc_sh.at[dst_v.at[k]], ssem[b])

  for b in range(H):
    _gather(b, b)

  @pl.loop(0, NCH, step=RING)
  def _(j):
    for b in range(RING):
      k = j + b

      @pl.when(k >= H)
      def _():
        _scat_desc(k - H, (b + H) % RING).wait()

      @pl.when(k + H < NCH)
      def _():
        _gather(k + H, (b + H) % RING)

      pltpu.make_async_copy(h_hbm.at[src_v.at[k]], rows[b], gsem[b]).wait()
      pltpu.async_copy(rows[b], acc_sh.at[dst_v.at[k]], ssem[b], add=True)

  for t in range(H):
    _scat_desc(NCH - H + t, (NCH - H + t) % RING).wait()
  plsc.subcore_barrier()
  pltpu.sync_copy(acc_sh.at[pl.ds(r0, ROWS_PER_TILE)],
                  out.at[cid, pl.ds(r0, ROWS_PER_TILE)])


_agg_call = pl.kernel(
    _agg_body,
    out_type=jax.ShapeDtypeStruct((NC, N_PAD, HIDDEN), jnp.float32),
    mesh=_MESH,
    compiler_params=_SC_PARAMS,
    scratch_types=[
        pltpu.VMEM((NCH, CHUNK), jnp.int32),
        pltpu.VMEM((NCH, CHUNK), jnp.int32),
    ] + [pltpu.VMEM((CHUNK, HIDDEN), jnp.float32)] * RING + [
        pltpu.VMEM_SHARED((N_PAD, HIDDEN), jnp.float32),
    ] + [pltpu.SemaphoreType.DMA] * (2 * RING),
)


def _tc1_body(degp, x_ref, w_ref, h_out, dis_out):
  deg = degp[0, :, 0:1] + degp[1, :, 0:1] + 1.0
  dis = lax.rsqrt(deg)
  h = jnp.dot(x_ref[...], w_ref[...], preferred_element_type=jnp.float32)
  hp = jnp.concatenate(
      [h, jnp.zeros((N_PAD - N_NODES, HIDDEN), jnp.float32)], axis=0)
  h_out[...] = hp * dis
  dis_out[...] = dis


def _tc2_body(part, hs_ref, dis_ref, b_ref, w_ref, out_ref):
  dis = dis_ref[...]
  s = part[0] + part[1] + hs_ref[...]
  h = jnp.maximum(dis * s + b_ref[...], 0.0)
  out_ref[...] = jnp.dot(
      h, w_ref[...], preferred_element_type=jnp.float32) * dis


def _tc3_body(part, hs_ref, dis_ref, b_ref, batch_ref, wout_ref, bout_ref,
              out_ref):
  dis = dis_ref[...]
  s = part[0] + part[1] + hs_ref[...]
  h2 = jnp.maximum(dis * s + b_ref[...], 0.0)
  gids = lax.broadcasted_iota(jnp.int32, (N_GRAPHS, N_PAD), 0)
  mask = (gids == batch_ref[...]).astype(jnp.float32)
  sums = jnp.dot(mask, h2, preferred_element_type=jnp.float32)
  cnt = jnp.sum(mask, axis=1, keepdims=True)
  g = sums / jnp.maximum(cnt, 1.0)
  out_ref[...] = jnp.dot(
      g, wout_ref[...], preferred_element_type=jnp.float32) + bout_ref[...]


def kernel(x, edge_index, batch, W1, b1, W2, b2, Wout, bout):
  src = edge_index[0].astype(jnp.int32)
  dst = edge_index[1].astype(jnp.int32)
  pad = E_PAD - N_EDGES
  pad_idx = N_NODES + (jnp.arange(pad, dtype=jnp.int32) % (N_PAD - N_NODES))
  src3 = jnp.concatenate([src, pad_idx]).reshape(NW, NCH, CHUNK)
  dst3 = jnp.concatenate([dst, pad_idx]).reshape(NW, NCH, CHUNK)

  zeros16 = jnp.zeros((N_PAD, 16), jnp.float32)
  zeros64 = jnp.zeros((N_PAD, HIDDEN), jnp.float32)
  ones_v = jnp.ones((CHUNK, 16), jnp.float32)
  batch_p = jnp.full((1, N_PAD), N_GRAPHS, jnp.int32).at[0, :N_NODES].set(
      batch.astype(jnp.int32))
  wout_p = jnp.zeros((HIDDEN, 128), jnp.float32).at[:, :2].set(Wout)
  bout_p = jnp.zeros((1, 128), jnp.float32).at[0, :2].set(bout)

  degp = _deg_call(dst3, ones_v, zeros16)

  h1s, dis = pl.pallas_call(
      _tc1_body,
      out_shape=(
          jax.ShapeDtypeStruct((N_PAD, HIDDEN), jnp.float32),
          jax.ShapeDtypeStruct((N_PAD, 1), jnp.float32),
      ),
  )(degp, x, W1)

  part1 = _agg_call(h1s, src3, dst3, zeros64)

  h2s = pl.pallas_call(
      _tc2_body,
      out_shape=jax.ShapeDtypeStruct((N_PAD, HIDDEN), jnp.float32),
  )(part1, h1s, dis, b1.reshape(1, HIDDEN), W2)

  part2 = _agg_call(h2s, src3, dst3, zeros64)

  outp = pl.pallas_call(
      _tc3_body,
      out_shape=jax.ShapeDtypeStruct((N_GRAPHS, 128), jnp.float32),
  )(part2, h2s, dis, b2.reshape(1, HIDDEN), batch_p, wout_p, bout_p)

  return outp[:, :2]


# speedup vs baseline: 46.1678x; 1.0366x over previous
import functools

import jax
import jax.numpy as jnp
from jax import lax
from jax.experimental import pallas as pl
from jax.experimental.pallas import tpu as pltpu
from jax.experimental.pallas import tpu_sc as plsc

N_NODES = 10000
N_EDGES = 320000
D_FEAT = 128
HIDDEN = 64
N_GRAPHS = 64

NC = 2
NS = 16
NW = NC * NS
CHUNK = 128
EPT = N_EDGES // NW
NCH = 80
TPT = NCH * CHUNK
RING = 8
ROWS_PER_TILE = 632
N_PAD = ROWS_PER_TILE * NS

_MESH = plsc.VectorSubcoreMesh(
    core_axis_name="c", subcore_axis_name="s", num_cores=NC, num_subcores=NS)


def _fill_dummy_tail(idx_v):
  lane = lax.iota(jnp.int32, 16)
  for i in range((TPT - EPT) // 16):
    off = i * 16
    idx_v[pl.ds(EPT + off, 16)] = N_NODES + (off + lane) % (N_PAD - N_NODES)


def _deg_body(ei, ones_v_hbm, zeros16, out, ones_v, dst_v, deg_sh, sem):
  cid = lax.axis_index("c")
  sid = lax.axis_index("s")
  wid = cid * NS + sid
  r0 = sid * ROWS_PER_TILE
  pltpu.sync_copy(zeros16.at[pl.ds(r0, ROWS_PER_TILE)],
                  deg_sh.at[pl.ds(r0, ROWS_PER_TILE)])
  pltpu.sync_copy(ones_v_hbm, ones_v)
  pltpu.sync_copy(ei.at[1, pl.ds(wid * EPT, EPT)], dst_v.at[pl.ds(0, EPT)])
  _fill_dummy_tail(dst_v)
  plsc.subcore_barrier()

  def _deg_desc(j):
    return pltpu.make_async_copy(
        ones_v, deg_sh.at[dst_v.at[pl.ds(j * CHUNK, CHUNK)]], sem)

  @pl.loop(0, NCH)
  def _(j):
    pltpu.async_copy(
        ones_v, deg_sh.at[dst_v.at[pl.ds(j * CHUNK, CHUNK)]], sem, add=True)

  @pl.loop(0, NCH)
  def _(j):
    _deg_desc(j).wait()

  plsc.subcore_barrier()
  pltpu.sync_copy(deg_sh.at[pl.ds(r0, ROWS_PER_TILE)],
                  out.at[cid, pl.ds(r0, ROWS_PER_TILE)])


_SC_PARAMS = pltpu.CompilerParams(use_tc_tiling_on_sc=False)

_deg_call = pl.kernel(
    _deg_body,
    out_type=jax.ShapeDtypeStruct((NC, N_PAD, 16), jnp.float32),
    mesh=_MESH,
    compiler_params=_SC_PARAMS,
    scratch_types=[
        pltpu.VMEM((CHUNK, 16), jnp.float32),
        pltpu.VMEM((TPT,), jnp.int32),
        pltpu.VMEM_SHARED((N_PAD, 16), jnp.float32),
        pltpu.SemaphoreType.DMA,
    ],
)


def _agg_body(h_hbm, ei, zeros64, out, src_v, dst_v, *bufs):
  rows = bufs[:RING]
  acc_sh = bufs[RING]
  (gsem0, gsem1, gsem2, gsem3, gsem4, gsem5, gsem6, gsem7,
   ssem0, ssem1, ssem2, ssem3, ssem4, ssem5, ssem6, ssem7) = bufs[RING + 1:]
  cid = lax.axis_index("c")
  sid = lax.axis_index("s")
  wid = cid * NS + sid
  r0 = sid * ROWS_PER_TILE
  pltpu.sync_copy(zeros64.at[pl.ds(r0, ROWS_PER_TILE)],
                  acc_sh.at[pl.ds(r0, ROWS_PER_TILE)])
  pltpu.sync_copy(ei.at[0, pl.ds(wid * EPT, EPT)], src_v.at[pl.ds(0, EPT)])
  pltpu.sync_copy(ei.at[1, pl.ds(wid * EPT, EPT)], dst_v.at[pl.ds(0, EPT)])
  _fill_dummy_tail(src_v)
  _fill_dummy_tail(dst_v)
  plsc.subcore_barrier()

  gsem = (gsem0, gsem1, gsem2, gsem3, gsem4, gsem5, gsem6, gsem7)
  ssem = (ssem0, ssem1, ssem2, ssem3, ssem4, ssem5, ssem6, ssem7)
  H = RING // 2

  def _gather(k, b):
    pltpu.async_copy(
        h_hbm.at[src_v.at[pl.ds(k * CHUNK, CHUNK)]], rows[b], gsem[b])

  def _scat_desc(k, b):
    return pltpu.make_async_copy(
        rows[b], acc_sh.at[dst_v.at[pl.ds(k * CHUNK, CHUNK)]], ssem[b])

  for b in range(H):
    _gather(b, b)

  @pl.loop(0, NCH, step=RING)
  def _(j):
    for b in range(RING):
      k = j + b

      @pl.when(k >= H)
      def _():
        _scat_desc(k - H, (b + H) % RING).wait()

      @pl.when(k + H < NCH)
      def _():
        _gather(k + H, (b + H) % RING)

      pltpu.make_async_copy(
          h_hbm.at[src_v.at[pl.ds(k * CHUNK, CHUNK)]], rows[b],
          gsem[b]).wait()
      pltpu.async_copy(
          rows[b], acc_sh.at[dst_v.at[pl.ds(k * CHUNK, CHUNK)]], ssem[b],
          add=True)

  for t in range(H):
    _scat_desc(NCH - H + t, (NCH - H + t) % RING).wait()
  plsc.subcore_barrier()
  pltpu.sync_copy(acc_sh.at[pl.ds(r0, ROWS_PER_TILE)],
                  out.at[cid, pl.ds(r0, ROWS_PER_TILE)])


_agg_call = pl.kernel(
    _agg_body,
    out_type=jax.ShapeDtypeStruct((NC, N_PAD, HIDDEN), jnp.float32),
    mesh=_MESH,
    compiler_params=_SC_PARAMS,
    scratch_types=[
        pltpu.VMEM((TPT,), jnp.int32),
        pltpu.VMEM((TPT,), jnp.int32),
    ] + [pltpu.VMEM((CHUNK, HIDDEN), jnp.float32)] * RING + [
        pltpu.VMEM_SHARED((N_PAD, HIDDEN), jnp.float32),
    ] + [pltpu.SemaphoreType.DMA] * (2 * RING),
)


def _tc1_body(degp, x_ref, w_ref, h_out, dis_out):
  deg = degp[0, :, 0:1] + degp[1, :, 0:1] + 1.0
  dis = lax.rsqrt(deg)
  h = jnp.dot(x_ref[...], w_ref[...], preferred_element_type=jnp.float32)
  hp = jnp.concatenate(
      [h, jnp.zeros((N_PAD - N_NODES, HIDDEN), jnp.float32)], axis=0)
  h_out[...] = hp * dis
  dis_out[...] = dis


def _tc2_body(part, hs_ref, dis_ref, b_ref, w_ref, out_ref):
  dis = dis_ref[...]
  s = part[0] + part[1] + hs_ref[...]
  h = jnp.maximum(dis * s + b_ref[...], 0.0)
  out_ref[...] = jnp.dot(
      h, w_ref[...], preferred_element_type=jnp.float32) * dis


def _tc3_body(part, hs_ref, dis_ref, b_ref, batch_ref, wout_ref, bout_ref,
              out_ref):
  dis = dis_ref[...]
  s = part[0] + part[1] + hs_ref[...]
  h2 = jnp.maximum(dis * s + b_ref[...], 0.0)
  gids = lax.broadcasted_iota(jnp.int32, (N_GRAPHS, N_PAD), 0)
  mask = (gids == batch_ref[...]).astype(jnp.float32)
  sums = jnp.dot(mask, h2, preferred_element_type=jnp.float32)
  cnt = jnp.sum(mask, axis=1, keepdims=True)
  g = sums / jnp.maximum(cnt, 1.0)
  out_ref[...] = jnp.dot(
      g, wout_ref[...], preferred_element_type=jnp.float32) + bout_ref[...]


def kernel(x, edge_index, batch, W1, b1, W2, b2, Wout, bout):
  ei = edge_index.astype(jnp.int32)
  zeros16 = jnp.zeros((N_PAD, 16), jnp.float32)
  zeros64 = jnp.zeros((N_PAD, HIDDEN), jnp.float32)
  ones_v = jnp.ones((CHUNK, 16), jnp.float32)
  batch_p = jnp.full((1, N_PAD), N_GRAPHS, jnp.int32).at[0, :N_NODES].set(
      batch.astype(jnp.int32))
  wout_p = jnp.zeros((HIDDEN, 128), jnp.float32).at[:, :2].set(Wout)
  bout_p = jnp.zeros((1, 128), jnp.float32).at[0, :2].set(bout)

  degp = _deg_call(ei, ones_v, zeros16)

  h1s, dis = pl.pallas_call(
      _tc1_body,
      out_shape=(
          jax.ShapeDtypeStruct((N_PAD, HIDDEN), jnp.float32),
          jax.ShapeDtypeStruct((N_PAD, 1), jnp.float32),
      ),
  )(degp, x, W1)

  part1 = _agg_call(h1s, ei, zeros64)

  h2s = pl.pallas_call(
      _tc2_body,
      out_shape=jax.ShapeDtypeStruct((N_PAD, HIDDEN), jnp.float32),
  )(part1, h1s, dis, b1.reshape(1, HIDDEN), W2)

  part2 = _agg_call(h2s, ei, zeros64)

  outp = pl.pallas_call(
      _tc3_body,
      out_shape=jax.ShapeDtypeStruct((N_GRAPHS, 128), jnp.float32),
  )(part2, h2s, dis, b2.reshape(1, HIDDEN), batch_p, wout_p, bout_p)

  return outp[:, :2]


# speedup vs baseline: 57.1693x; 1.2383x over previous
import functools

import jax
import jax.numpy as jnp
from jax import lax
from jax.experimental import pallas as pl
from jax.experimental.pallas import tpu as pltpu
from jax.experimental.pallas import tpu_sc as plsc

N_NODES = 10000
N_EDGES = 320000
D_FEAT = 128
HIDDEN = 64
N_GRAPHS = 64

NC = 2
NS = 16
NW = NC * NS
CHUNK = 128
EPT = N_EDGES // NW
NCH = 80
TPT = NCH * CHUNK
RING = 8
ROWS_PER_TILE = 632
N_PAD = ROWS_PER_TILE * NS

_MESH = plsc.VectorSubcoreMesh(
    core_axis_name="c", subcore_axis_name="s", num_cores=NC, num_subcores=NS)


def _fill_dummy_tail(idx_v):
  lane = lax.iota(jnp.int32, 16)
  for i in range((TPT - EPT) // 16):
    off = i * 16
    idx_v[pl.ds(EPT + off, 16)] = N_NODES + (off + lane) % (N_PAD - N_NODES)


def _deg_body(ei, ones_v_hbm, zeros16, out, ones_v, dst_v, deg_sh, sem):
  cid = lax.axis_index("c")
  sid = lax.axis_index("s")
  wid = cid * NS + sid
  r0 = sid * ROWS_PER_TILE
  pltpu.sync_copy(zeros16.at[pl.ds(r0, ROWS_PER_TILE)],
                  deg_sh.at[pl.ds(r0, ROWS_PER_TILE)])
  pltpu.sync_copy(ones_v_hbm, ones_v)
  pltpu.sync_copy(ei.at[1, pl.ds(wid * EPT, EPT)], dst_v.at[pl.ds(0, EPT)])
  _fill_dummy_tail(dst_v)
  plsc.subcore_barrier()

  def _deg_desc(j):
    return pltpu.make_async_copy(
        ones_v, deg_sh.at[dst_v.at[pl.ds(j * CHUNK, CHUNK)]], sem)

  @pl.loop(0, NCH)
  def _(j):
    pltpu.async_copy(
        ones_v, deg_sh.at[dst_v.at[pl.ds(j * CHUNK, CHUNK)]], sem, add=True)

  @pl.loop(0, NCH)
  def _(j):
    _deg_desc(j).wait()

  plsc.subcore_barrier()
  pltpu.sync_copy(deg_sh.at[pl.ds(r0, ROWS_PER_TILE)],
                  out.at[cid, pl.ds(r0, ROWS_PER_TILE)])


_SC_PARAMS = pltpu.CompilerParams(use_tc_tiling_on_sc=False)

_deg_call = pl.kernel(
    _deg_body,
    out_type=jax.ShapeDtypeStruct((NC, N_PAD, 16), jnp.float32),
    mesh=_MESH,
    compiler_params=_SC_PARAMS,
    scratch_types=[
        pltpu.VMEM((CHUNK, 16), jnp.float32),
        pltpu.VMEM((TPT,), jnp.int32),
        pltpu.VMEM_SHARED((N_PAD, 16), jnp.float32),
        pltpu.SemaphoreType.DMA,
    ],
)


def _agg_body(h_hbm, ei, zeros64, out, src_v, dst_v, *bufs):
  rows = bufs[:RING]
  acc_sh = bufs[RING]
  (gsem0, gsem1, gsem2, gsem3, gsem4, gsem5, gsem6, gsem7,
   ssem0, ssem1, ssem2, ssem3, ssem4, ssem5, ssem6, ssem7) = bufs[RING + 1:]
  cid = lax.axis_index("c")
  sid = lax.axis_index("s")
  wid = cid * NS + sid
  r0 = sid * ROWS_PER_TILE
  pltpu.sync_copy(zeros64.at[pl.ds(r0, ROWS_PER_TILE)],
                  acc_sh.at[pl.ds(r0, ROWS_PER_TILE)])
  pltpu.sync_copy(ei.at[0, pl.ds(wid * EPT, EPT)], src_v.at[pl.ds(0, EPT)])
  pltpu.sync_copy(ei.at[1, pl.ds(wid * EPT, EPT)], dst_v.at[pl.ds(0, EPT)])
  _fill_dummy_tail(src_v)
  _fill_dummy_tail(dst_v)
  plsc.subcore_barrier()

  gsem = (gsem0, gsem1, gsem2, gsem3, gsem4, gsem5, gsem6, gsem7)
  ssem = (ssem0, ssem1, ssem2, ssem3, ssem4, ssem5, ssem6, ssem7)
  H = RING // 2

  def _gather(k, b):
    pltpu.async_copy(
        h_hbm.at[src_v.at[pl.ds(k * CHUNK, CHUNK)]], rows[b], gsem[b])

  def _scat_desc(k, b):
    return pltpu.make_async_copy(
        rows[b], acc_sh.at[dst_v.at[pl.ds(k * CHUNK, CHUNK)]], ssem[b])

  for b in range(H):
    _gather(b, b)

  @pl.loop(0, NCH, step=RING)
  def _(j):
    for b in range(RING):
      k = j + b

      @pl.when(k >= H)
      def _():
        _scat_desc(k - H, (b + H) % RING).wait()

      @pl.when(k + H < NCH)
      def _():
        _gather(k + H, (b + H) % RING)

      pltpu.make_async_copy(
          h_hbm.at[src_v.at[pl.ds(k * CHUNK, CHUNK)]], rows[b],
          gsem[b]).wait()
      pltpu.async_copy(
          rows[b], acc_sh.at[dst_v.at[pl.ds(k * CHUNK, CHUNK)]], ssem[b],
          add=True)

  for t in range(H):
    _scat_desc(NCH - H + t, (NCH - H + t) % RING).wait()
  plsc.subcore_barrier()
  pltpu.sync_copy(acc_sh.at[pl.ds(r0, ROWS_PER_TILE)],
                  out.at[cid, pl.ds(r0, ROWS_PER_TILE)])


_agg_call = pl.kernel(
    _agg_body,
    out_type=jax.ShapeDtypeStruct((NC, N_PAD, HIDDEN), jnp.float32),
    mesh=_MESH,
    compiler_params=_SC_PARAMS,
    scratch_types=[
        pltpu.VMEM((TPT,), jnp.int32),
        pltpu.VMEM((TPT,), jnp.int32),
    ] + [pltpu.VMEM((CHUNK, HIDDEN), jnp.float32)] * RING + [
        pltpu.VMEM_SHARED((N_PAD, HIDDEN), jnp.float32),
    ] + [pltpu.SemaphoreType.DMA] * (2 * RING),
)


NPP = N_PAD // 2
NDR = N_PAD // 8


def _disp(degp):
  d = degp[0] + degp[1]
  drep = jnp.broadcast_to(d[:, None, :], (NDR, 4, 128)).reshape(NPP, 128)
  q = lax.broadcasted_iota(jnp.int32, (NPP, 128), 0) % 4
  lane = lax.broadcasted_iota(jnp.int32, (NPP, 128), 1)
  dege = jnp.sum(jnp.where(lane == 32 * q, drep, 0.0), axis=1, keepdims=True)
  dego = jnp.sum(
      jnp.where(lane == 32 * q + 16, drep, 0.0), axis=1, keepdims=True)
  dise = lax.rsqrt(dege + 1.0)
  diso = lax.rsqrt(dego + 1.0)
  return jnp.where(lane < 64, dise, diso)


def _tc1_body(degp, xe_ref, xo_ref, w_ref, h_out):
  disp = _disp(degp)
  he = jnp.dot(xe_ref[...], w_ref[...], preferred_element_type=jnp.float32)
  ho = jnp.dot(xo_ref[...], w_ref[...], preferred_element_type=jnp.float32)
  hp = jnp.concatenate([he, ho], axis=1)
  hp = jnp.concatenate(
      [hp, jnp.zeros((NPP - N_NODES // 2, 128), jnp.float32)], axis=0)
  h_out[...] = hp * disp


def _tc2_body(part, hs_ref, degp, b_ref, w_ref, out_ref):
  disp = _disp(degp)
  s = part[0] + part[1] + hs_ref[...]
  h = jnp.maximum(disp * s + b_ref[...], 0.0)
  out_ref[...] = jnp.dot(
      h, w_ref[...], preferred_element_type=jnp.float32) * disp


def _tc3_body(part, hs_ref, degp, b_ref, be_ref, bo_ref, wout_ref, bout_ref,
              out_ref):
  disp = _disp(degp)
  s = part[0] + part[1] + hs_ref[...]
  h2 = jnp.maximum(disp * s + b_ref[...], 0.0)
  gids = lax.broadcasted_iota(jnp.int32, (N_GRAPHS, NPP), 0)
  ma = (gids == be_ref[...]).astype(jnp.float32)
  mb = (gids == bo_ref[...]).astype(jnp.float32)
  sums = (jnp.dot(ma, h2[:, :HIDDEN], preferred_element_type=jnp.float32)
          + jnp.dot(mb, h2[:, HIDDEN:], preferred_element_type=jnp.float32))
  cnt = (jnp.sum(ma, axis=1, keepdims=True)
         + jnp.sum(mb, axis=1, keepdims=True))
  g = sums / jnp.maximum(cnt, 1.0)
  out_ref[...] = jnp.dot(
      g, wout_ref[...], preferred_element_type=jnp.float32) + bout_ref[...]


def kernel(x, edge_index, batch, W1, b1, W2, b2, Wout, bout):
  ei = edge_index.astype(jnp.int32)
  zeros16 = jnp.zeros((N_PAD, 16), jnp.float32)
  zeros64 = jnp.zeros((N_PAD, HIDDEN), jnp.float32)
  ones_v = jnp.ones((CHUNK, 16), jnp.float32)
  batch_p = jnp.full((N_PAD,), N_GRAPHS, jnp.int32).at[:N_NODES].set(
      batch.astype(jnp.int32))
  b_even = batch_p[0::2].reshape(1, NPP)
  b_odd = batch_p[1::2].reshape(1, NPP)
  wout_p = jnp.zeros((HIDDEN, 128), jnp.float32).at[:, :2].set(Wout)
  bout_p = jnp.zeros((1, 128), jnp.float32).at[0, :2].set(bout)
  xe, xo = x[0::2], x[1::2]
  w2d = jnp.zeros((128, 128), jnp.float32)
  w2d = w2d.at[:HIDDEN, :HIDDEN].set(W2).at[HIDDEN:, HIDDEN:].set(W2)
  b1p = jnp.concatenate([b1, b1]).reshape(1, 128)
  b2p = jnp.concatenate([b2, b2]).reshape(1, 128)

  degp = _deg_call(ei, ones_v, zeros16).reshape(NC, NDR, 128)

  h1s = pl.pallas_call(
      _tc1_body,
      out_shape=jax.ShapeDtypeStruct((NPP, 128), jnp.float32),
  )(degp, xe, xo, W1)

  part1 = _agg_call(h1s.reshape(N_PAD, HIDDEN), ei, zeros64)

  h2s = pl.pallas_call(
      _tc2_body,
      out_shape=jax.ShapeDtypeStruct((NPP, 128), jnp.float32),
  )(part1.reshape(NC, NPP, 128), h1s, degp, b1p, w2d)

  part2 = _agg_call(h2s.reshape(N_PAD, HIDDEN), ei, zeros64)

  outp = pl.pallas_call(
      _tc3_body,
      out_shape=jax.ShapeDtypeStruct((N_GRAPHS, 128), jnp.float32),
  )(part2.reshape(NC, NPP, 128), h2s, degp, b2p, b_even, b_odd, wout_p,
    bout_p)

  return outp[:, :2]


# speedup vs baseline: 57.6771x; 1.0089x over previous
import functools

import jax
import jax.numpy as jnp
from jax import lax
from jax.experimental import pallas as pl
from jax.experimental.pallas import tpu as pltpu
from jax.experimental.pallas import tpu_sc as plsc

N_NODES = 10000
N_EDGES = 320000
D_FEAT = 128
HIDDEN = 64
N_GRAPHS = 64

NC = 2
NS = 16
NW = NC * NS
CHUNK = 128
EPT = N_EDGES // NW
NCH = 80
TPT = NCH * CHUNK
RING = 8
ROWS_PER_TILE = 632
N_PAD = ROWS_PER_TILE * NS

_MESH = plsc.VectorSubcoreMesh(
    core_axis_name="c", subcore_axis_name="s", num_cores=NC, num_subcores=NS)


def _fill_dummy_tail(idx_v):
  lane = lax.iota(jnp.int32, 16)
  for i in range((TPT - EPT) // 16):
    off = i * 16
    idx_v[pl.ds(EPT + off, 16)] = N_NODES + (off + lane) % (N_PAD - N_NODES)


def _deg_body(ei, ones_v_hbm, zeros16, out, ones_v, dst_v, deg_sh, sem):
  cid = lax.axis_index("c")
  sid = lax.axis_index("s")
  wid = cid * NS + sid
  r0 = sid * ROWS_PER_TILE
  pltpu.sync_copy(zeros16.at[pl.ds(r0, ROWS_PER_TILE)],
                  deg_sh.at[pl.ds(r0, ROWS_PER_TILE)])
  pltpu.sync_copy(ones_v_hbm, ones_v)
  pltpu.sync_copy(ei.at[1, pl.ds(wid * EPT, EPT)], dst_v.at[pl.ds(0, EPT)])
  _fill_dummy_tail(dst_v)
  plsc.subcore_barrier()

  def _deg_desc(j):
    return pltpu.make_async_copy(
        ones_v, deg_sh.at[dst_v.at[pl.ds(j * CHUNK, CHUNK)]], sem)

  @pl.loop(0, NCH)
  def _(j):
    pltpu.async_copy(
        ones_v, deg_sh.at[dst_v.at[pl.ds(j * CHUNK, CHUNK)]], sem, add=True)

  @pl.loop(0, NCH)
  def _(j):
    _deg_desc(j).wait()

  plsc.subcore_barrier()
  pltpu.sync_copy(deg_sh.at[pl.ds(r0, ROWS_PER_TILE)],
                  out.at[cid, pl.ds(r0, ROWS_PER_TILE)])


_SC_PARAMS = pltpu.CompilerParams(use_tc_tiling_on_sc=False)

_deg_call = pl.kernel(
    _deg_body,
    out_type=jax.ShapeDtypeStruct((NC, N_PAD, 16), jnp.float32),
    mesh=_MESH,
    compiler_params=_SC_PARAMS,
    scratch_types=[
        pltpu.VMEM((CHUNK, 16), jnp.float32),
        pltpu.VMEM((TPT,), jnp.int32),
        pltpu.VMEM_SHARED((N_PAD, 16), jnp.float32),
        pltpu.SemaphoreType.DMA,
    ],
)


def _agg_body(h_hbm, ei, zeros64, out, src_v, dst_v, *bufs):
  rows = bufs[:RING]
  acc_sh = bufs[RING]
  (gsem0, gsem1, gsem2, gsem3, gsem4, gsem5, gsem6, gsem7,
   ssem0, ssem1, ssem2, ssem3, ssem4, ssem5, ssem6, ssem7) = bufs[RING + 1:]
  cid = lax.axis_index("c")
  sid = lax.axis_index("s")
  wid = cid * NS + sid
  r0 = sid * ROWS_PER_TILE
  pltpu.sync_copy(zeros64.at[pl.ds(r0, ROWS_PER_TILE)],
                  acc_sh.at[pl.ds(r0, ROWS_PER_TILE)])
  pltpu.sync_copy(ei.at[0, pl.ds(wid * EPT, EPT)], src_v.at[pl.ds(0, EPT)])
  pltpu.sync_copy(ei.at[1, pl.ds(wid * EPT, EPT)], dst_v.at[pl.ds(0, EPT)])
  _fill_dummy_tail(src_v)
  _fill_dummy_tail(dst_v)
  plsc.subcore_barrier()

  gsem = (gsem0, gsem1, gsem2, gsem3, gsem4, gsem5, gsem6, gsem7)
  ssem = (ssem0, ssem1, ssem2, ssem3, ssem4, ssem5, ssem6, ssem7)
  H = RING // 2

  def _gather(k, b):
    pltpu.async_copy(
        h_hbm.at[src_v.at[pl.ds(k * CHUNK, CHUNK)]], rows[b], gsem[b])

  def _scat_desc(k, b):
    return pltpu.make_async_copy(
        rows[b], acc_sh.at[dst_v.at[pl.ds(k * CHUNK, CHUNK)]], ssem[b])

  for b in range(H):
    _gather(b, b)

  @pl.loop(0, NCH, step=RING)
  def _(j):
    for b in range(RING):
      k = j + b

      @pl.when(k >= H)
      def _():
        _scat_desc(k - H, (b + H) % RING).wait()

      @pl.when(k + H < NCH)
      def _():
        _gather(k + H, (b + H) % RING)

      pltpu.make_async_copy(
          h_hbm.at[src_v.at[pl.ds(k * CHUNK, CHUNK)]], rows[b],
          gsem[b]).wait()
      pltpu.async_copy(
          rows[b], acc_sh.at[dst_v.at[pl.ds(k * CHUNK, CHUNK)]], ssem[b],
          add=True)

  for t in range(H):
    _scat_desc(NCH - H + t, (NCH - H + t) % RING).wait()
  plsc.subcore_barrier()
  pltpu.sync_copy(acc_sh.at[pl.ds(r0, ROWS_PER_TILE)],
                  out.at[cid, pl.ds(r0, ROWS_PER_TILE)])


_agg_call = pl.kernel(
    _agg_body,
    out_type=jax.ShapeDtypeStruct((NC, N_PAD, HIDDEN), jnp.float32),
    mesh=_MESH,
    compiler_params=_SC_PARAMS,
    scratch_types=[
        pltpu.VMEM((TPT,), jnp.int32),
        pltpu.VMEM((TPT,), jnp.int32),
    ] + [pltpu.VMEM((CHUNK, HIDDEN), jnp.float32)] * RING + [
        pltpu.VMEM_SHARED((N_PAD, HIDDEN), jnp.float32),
    ] + [pltpu.SemaphoreType.DMA] * (2 * RING),
)


NPP = N_PAD // 2
NDR = N_PAD // 8


def _disp(degp):
  d = degp[0] + degp[1]
  drep = jnp.broadcast_to(d[:, None, :], (NDR, 4, 128)).reshape(NPP, 128)
  q = lax.broadcasted_iota(jnp.int32, (NPP, 128), 0) % 4
  lane = lax.broadcasted_iota(jnp.int32, (NPP, 128), 1)
  dege = jnp.sum(jnp.where(lane == 32 * q, drep, 0.0), axis=1, keepdims=True)
  dego = jnp.sum(
      jnp.where(lane == 32 * q + 16, drep, 0.0), axis=1, keepdims=True)
  dise = lax.rsqrt(dege + 1.0)
  diso = lax.rsqrt(dego + 1.0)
  return jnp.where(lane < 64, dise, diso)


def _tc1_body(degp, xe_ref, xo_ref, w_ref, h_out):
  disp = _disp(degp)
  he = jnp.dot(xe_ref[...], w_ref[...], preferred_element_type=jnp.float32)
  ho = jnp.dot(xo_ref[...], w_ref[...], preferred_element_type=jnp.float32)
  hp = jnp.concatenate([he, ho], axis=1)
  hp = jnp.concatenate(
      [hp, jnp.zeros((NPP - N_NODES // 2, 128), jnp.float32)], axis=0)
  h_out[...] = hp * disp


def _tc2_body(part, hs_ref, degp, b_ref, w_ref, out_ref):
  disp = _disp(degp)
  s = part[0] + part[1] + hs_ref[...]
  h = jnp.maximum(disp * s + b_ref[...], 0.0)
  out_ref[...] = jnp.dot(
      h, w_ref[...], preferred_element_type=jnp.float32) * disp


def _tc3_body(part, hs_ref, degp, b_ref, be_ref, bo_ref, wout_ref, bout_ref,
              out_ref):
  disp = _disp(degp)
  s = part[0] + part[1] + hs_ref[...]
  h2 = jnp.maximum(disp * s + b_ref[...], 0.0)
  gids = lax.broadcasted_iota(jnp.int32, (N_GRAPHS, NPP), 0)
  ma = (gids == be_ref[...]).astype(jnp.float32)
  mb = (gids == bo_ref[...]).astype(jnp.float32)
  sums = (jnp.dot(ma, h2[:, :HIDDEN], preferred_element_type=jnp.float32)
          + jnp.dot(mb, h2[:, HIDDEN:], preferred_element_type=jnp.float32))
  cnt = (jnp.sum(ma, axis=1, keepdims=True)
         + jnp.sum(mb, axis=1, keepdims=True))
  g = sums / jnp.maximum(cnt, 1.0)
  out_ref[...] = jnp.dot(
      g, wout_ref[...], preferred_element_type=jnp.float32) + bout_ref[...]


def kernel(x, edge_index, batch, W1, b1, W2, b2, Wout, bout):
  ei = edge_index.astype(jnp.int32)
  zeros16 = jnp.zeros((N_PAD, 16), jnp.float32)
  zeros64 = jnp.zeros((N_PAD, HIDDEN), jnp.float32)
  ones_v = jnp.ones((CHUNK, 16), jnp.float32)
  batch_p = jnp.full((N_PAD,), N_GRAPHS, jnp.int32).at[:N_NODES].set(
      batch.astype(jnp.int32))
  b_even = batch_p[0::2].reshape(1, NPP)
  b_odd = batch_p[1::2].reshape(1, NPP)
  xe, xo = x[0::2], x[1::2]
  w2d = jnp.zeros((128, 128), jnp.float32)
  w2d = w2d.at[:HIDDEN, :HIDDEN].set(W2).at[HIDDEN:, HIDDEN:].set(W2)
  b1p = jnp.concatenate([b1, b1]).reshape(1, 128)
  b2p = jnp.concatenate([b2, b2]).reshape(1, 128)

  degp = _deg_call(ei, ones_v, zeros16).reshape(NC, NDR, 128)

  h1s = pl.pallas_call(
      _tc1_body,
      out_shape=jax.ShapeDtypeStruct((NPP, 128), jnp.float32),
  )(degp, xe, xo, W1)

  part1 = _agg_call(h1s.reshape(N_PAD, HIDDEN), ei, zeros64)

  h2s = pl.pallas_call(
      _tc2_body,
      out_shape=jax.ShapeDtypeStruct((NPP, 128), jnp.float32),
  )(part1.reshape(NC, NPP, 128), h1s, degp, b1p, w2d)

  part2 = _agg_call(h2s.reshape(N_PAD, HIDDEN), ei, zeros64)

  return pl.pallas_call(
      _tc3_body,
      out_shape=jax.ShapeDtypeStruct((N_GRAPHS, 2), jnp.float32),
  )(part2.reshape(NC, NPP, 128), h2s, degp, b2p, b_even, b_odd, Wout,
    bout.reshape(1, 2))
